# baseline (speedup 1.0000x reference)
import functools

import jax
import jax.numpy as jnp
from jax import lax
from jax.experimental import pallas as pl
from jax.experimental.pallas import tpu as pltpu
from jax.experimental.pallas import tpu_sc as plsc

DIM = 64
LANES = 16
K = DIM // LANES
NC, NS = 2, 16
NW = NC * NS
CHUNK = 128
GROUPS = CHUNK // LANES
EPS = 1e-8


def _rsqrt(t):
    i = lax.bitcast_convert_type(t, jnp.int32)
    i = jnp.int32(0x5F3759DF) - lax.shift_right_logical(i, 1)
    y = lax.bitcast_convert_type(i, jnp.float32)
    for _ in range(3):
        y = y * (1.5 - 0.5 * t * y * y)
    return y


def _body(tok_hbm, table_hbm, gamma_hbm, beta_hbm, out_hbm,
          idx_v, rows_v, gb_v, scr_v, gsem):
    wid = lax.axis_index("s") * NC + lax.axis_index("c")
    n_chunks = idx_v.shape[0]
    per_w = n_chunks * CHUNK

    pltpu.sync_copy(tok_hbm.at[wid], idx_v)
    pltpu.sync_copy(gamma_hbm, gb_v.at[pl.ds(0, DIM)])
    pltpu.sync_copy(beta_hbm, gb_v.at[pl.ds(DIM, DIM)])

    iota = lax.iota(jnp.int32, LANES)
    g_regs = [gb_v[pl.ds(k * LANES, LANES)] for k in range(K)]
    b_regs = [gb_v[pl.ds(DIM + k * LANES, LANES)] for k in range(K)]

    def chunk_body(c, carry):
        pltpu.async_copy(table_hbm.at[idx_v.at[c]], rows_v, gsem).wait()

        def group_body(g, gcarry):
            row0 = g * LANES
            rowidx = row0 + iota
            tokv = idx_v[c, pl.ds(row0, LANES)]

            s = jnp.zeros((LANES,), jnp.float32)
            s2 = jnp.zeros((LANES,), jnp.float32)
            for j in range(DIM):
                v = plsc.load_gather(
                    rows_v, [rowidx, jnp.full((LANES,), j, jnp.int32)])
                s = s + v
                s2 = s2 + v * v

            mean = s * (1.0 / DIM)
            var = s2 * (1.0 / DIM) - mean * mean
            inv = _rsqrt(var + EPS)
            m = jnp.where(tokv != 0, 1.0, 0.0).astype(jnp.float32)
            a = inv * m

            scr_v[pl.ds(LANES, LANES)] = mean
            scr_v[pl.ds(2 * LANES, LANES)] = a
            scr_v[pl.ds(3 * LANES, LANES)] = m

            for r in range(LANES):
                mean_b = plsc.load_gather(
                    scr_v, [jnp.full((LANES,), LANES + r, jnp.int32)])
                a_b = plsc.load_gather(
                    scr_v, [jnp.full((LANES,), 2 * LANES + r, jnp.int32)])
                m_b = plsc.load_gather(
                    scr_v, [jnp.full((LANES,), 3 * LANES + r, jnp.int32)])
                row = row0 + r
                for k in range(K):
                    x = rows_v[row, pl.ds(k * LANES, LANES)]
                    y = (x - mean_b) * a_b * g_regs[k] + b_regs[k] * m_b
                    rows_v[row, pl.ds(k * LANES, LANES)] = y
            return gcarry

        lax.fori_loop(0, GROUPS, group_body, 0)
        base = wid * per_w + c * CHUNK
        pltpu.sync_copy(rows_v, out_hbm.at[pl.ds(base, CHUNK)])
        return carry

    lax.fori_loop(0, n_chunks, chunk_body, 0)


def kernel(tokens, table, gamma, beta):
    Bt, Lt = tokens.shape
    N = Bt * Lt
    per_w = N // NW
    n_chunks = per_w // CHUNK
    tok3 = tokens.reshape(NW, n_chunks, CHUNK).astype(jnp.int32)

    mesh = plsc.VectorSubcoreMesh(core_axis_name="c", subcore_axis_name="s")
    sc_call = pl.kernel(
        _body,
        out_type=jax.ShapeDtypeStruct((N, DIM), jnp.float32),
        mesh=mesh,
        compiler_params=pltpu.CompilerParams(
            needs_layout_passes=False, use_tc_tiling_on_sc=False),
        scratch_types=[
            pltpu.VMEM((n_chunks, CHUNK), jnp.int32),
            pltpu.VMEM((CHUNK, DIM), jnp.float32),
            pltpu.VMEM((2 * DIM,), jnp.float32),
            pltpu.VMEM((4 * LANES,), jnp.float32),
            pltpu.SemaphoreType.DMA,
        ],
    )
    out = sc_call(tok3, table, gamma, beta)
    return out.reshape(Bt, Lt, DIM)


# speedup vs baseline: 1.0433x; 1.0433x over previous
import functools

import jax
import jax.numpy as jnp
from jax import lax
from jax.experimental import pallas as pl
from jax.experimental.pallas import tpu as pltpu
from jax.experimental.pallas import tpu_sc as plsc

DIM = 64
LANES = 16
K = DIM // LANES
NC, NS = 2, 16
NW = NC * NS
CHUNK = 128
GROUPS = CHUNK // LANES
EPS = 1e-8


def _rsqrt(t):
    i = lax.bitcast_convert_type(t, jnp.int32)
    i = jnp.int32(0x5F3759DF) - lax.shift_right_logical(i, 1)
    y = lax.bitcast_convert_type(i, jnp.float32)
    for _ in range(3):
        y = y * (1.5 - 0.5 * t * y * y)
    return y


def _body(tok_hbm, table_hbm, gamma_hbm, beta_hbm, out_hbm,
          idx_v, rows0_v, rows1_v, gb_v, scr_v,
          gsem0, gsem1, osem0, osem1):
    wid = lax.axis_index("s") * NC + lax.axis_index("c")
    n_chunks = idx_v.shape[0]
    per_w = n_chunks * CHUNK

    pltpu.sync_copy(tok_hbm.at[wid], idx_v)
    pltpu.sync_copy(gamma_hbm, gb_v.at[pl.ds(0, DIM)])
    pltpu.sync_copy(beta_hbm, gb_v.at[pl.ds(DIM, DIM)])

    iota = lax.iota(jnp.int32, LANES)
    g_regs = [gb_v[pl.ds(k * LANES, LANES)] for k in range(K)]
    b_regs = [gb_v[pl.ds(DIM + k * LANES, LANES)] for k in range(K)]

    rows = (rows0_v, rows1_v)
    gsems = (gsem0, gsem1)
    osems = (osem0, osem1)

    def gather_start(c, b):
        pltpu.make_async_copy(
            table_hbm.at[idx_v.at[c]], rows[b], gsems[b]).start()

    def out_start(c, b):
        base = wid * per_w + c * CHUNK
        pltpu.make_async_copy(
            rows[b], out_hbm.at[pl.ds(base, CHUNK)], osems[b]).start()

    def out_wait(b):
        pltpu.make_async_copy(
            rows[b], out_hbm.at[pl.ds(0, CHUNK)], osems[b]).wait()

    gather_start(0, 0)

    def process(c, b, rows_v):
        @pl.when(c >= 1)
        def _():
            out_wait(1 - b)

        @pl.when(c + 1 < n_chunks)
        def _():
            gather_start(c + 1, 1 - b)

        pltpu.make_async_copy(
            table_hbm.at[idx_v.at[c]], rows_v, gsems[b]).wait()

        def group_body(g, gcarry):
            row0 = g * LANES
            rowidx = row0 + iota
            tokv = idx_v[c, pl.ds(row0, LANES)]

            s = jnp.zeros((LANES,), jnp.float32)
            s2 = jnp.zeros((LANES,), jnp.float32)
            for j in range(DIM):
                v = plsc.load_gather(
                    rows_v, [rowidx, jnp.full((LANES,), j, jnp.int32)])
                s = s + v
                s2 = s2 + v * v

            mean = s * (1.0 / DIM)
            var = s2 * (1.0 / DIM) - mean * mean
            inv = _rsqrt(var + EPS)
            m = jnp.where(tokv != 0, 1.0, 0.0).astype(jnp.float32)
            a = inv * m

            scr_v[pl.ds(LANES, LANES)] = mean
            scr_v[pl.ds(2 * LANES, LANES)] = a
            scr_v[pl.ds(3 * LANES, LANES)] = m

            for r in range(LANES):
                mean_b = plsc.load_gather(
                    scr_v, [jnp.full((LANES,), LANES + r, jnp.int32)])
                a_b = plsc.load_gather(
                    scr_v, [jnp.full((LANES,), 2 * LANES + r, jnp.int32)])
                m_b = plsc.load_gather(
                    scr_v, [jnp.full((LANES,), 3 * LANES + r, jnp.int32)])
                row = row0 + r
                for k in range(K):
                    x = rows_v[row, pl.ds(k * LANES, LANES)]
                    y = (x - mean_b) * a_b * g_regs[k] + b_regs[k] * m_b
                    rows_v[row, pl.ds(k * LANES, LANES)] = y
            return gcarry

        lax.fori_loop(0, GROUPS, group_body, 0)
        out_start(c, b)

    def pair_body(cc, carry):
        process(2 * cc, 0, rows0_v)
        process(2 * cc + 1, 1, rows1_v)
        return carry

    lax.fori_loop(0, n_chunks // 2, pair_body, 0)
    out_wait(1)


def kernel(tokens, table, gamma, beta):
    Bt, Lt = tokens.shape
    N = Bt * Lt
    per_w = N // NW
    n_chunks = per_w // CHUNK
    tok3 = tokens.reshape(NW, n_chunks, CHUNK).astype(jnp.int32)

    mesh = plsc.VectorSubcoreMesh(core_axis_name="c", subcore_axis_name="s")
    sc_call = pl.kernel(
        _body,
        out_type=jax.ShapeDtypeStruct((N, DIM), jnp.float32),
        mesh=mesh,
        compiler_params=pltpu.CompilerParams(
            needs_layout_passes=False, use_tc_tiling_on_sc=False),
        scratch_types=[
            pltpu.VMEM((n_chunks, CHUNK), jnp.int32),
            pltpu.VMEM((CHUNK, DIM), jnp.float32),
            pltpu.VMEM((CHUNK, DIM), jnp.float32),
            pltpu.VMEM((2 * DIM,), jnp.float32),
            pltpu.VMEM((4 * LANES,), jnp.float32),
            pltpu.SemaphoreType.DMA,
            pltpu.SemaphoreType.DMA,
            pltpu.SemaphoreType.DMA,
            pltpu.SemaphoreType.DMA,
        ],
    )
    out = sc_call(tok3, table, gamma, beta)
    return out.reshape(Bt, Lt, DIM)


# speedup vs baseline: 1.1981x; 1.1483x over previous
import functools

import jax
import jax.numpy as jnp
from jax import lax
from jax.experimental import pallas as pl
from jax.experimental.pallas import tpu as pltpu
from jax.experimental.pallas import tpu_sc as plsc

DIM = 64
LANES = 16
K = DIM // LANES
NC, NS = 2, 16
NW = NC * NS
CHUNK = 128
GROUPS = CHUNK // LANES
EPS = 1e-8


def _rsqrt(t):
    i = lax.bitcast_convert_type(t, jnp.int32)
    i = jnp.int32(0x5F3759DF) - lax.shift_right_logical(i, 1)
    y = lax.bitcast_convert_type(i, jnp.float32)
    for _ in range(2):
        y = y * (1.5 - 0.5 * t * y * y)
    return y


def _body(tok_hbm, table_hbm, gamma_hbm, beta_hbm, out_hbm,
          idx_v, rows0_v, rows1_v, out0_v, out1_v, gb_v, scr_v,
          gsem0, gsem1, osem0, osem1):
    wid = lax.axis_index("s") * NC + lax.axis_index("c")
    n_chunks = idx_v.shape[0]
    per_w = n_chunks * CHUNK

    pltpu.sync_copy(tok_hbm.at[wid], idx_v)
    pltpu.sync_copy(gamma_hbm, gb_v.at[pl.ds(0, DIM)])
    pltpu.sync_copy(beta_hbm, gb_v.at[pl.ds(DIM, DIM)])

    iota = lax.iota(jnp.int32, LANES)
    g_regs = [gb_v[pl.ds(k * LANES, LANES)] for k in range(K)]
    b_regs = [gb_v[pl.ds(DIM + k * LANES, LANES)] for k in range(K)]

    rows = (rows0_v, rows1_v)
    outs = (out0_v, out1_v)
    gsems = (gsem0, gsem1)
    osems = (osem0, osem1)

    def gather_start(c, b):
        pltpu.make_async_copy(
            table_hbm.at[idx_v.at[c]], rows[b], gsems[b]).start()

    def out_start(c, b):
        base = wid * per_w + c * CHUNK
        pltpu.make_async_copy(
            outs[b], out_hbm.at[pl.ds(base, CHUNK)], osems[b]).start()

    def out_wait(b):
        pltpu.make_async_copy(
            outs[b], out_hbm.at[pl.ds(0, CHUNK)], osems[b]).wait()

    gather_start(0, 0)

    def process(c, b, rows_v, out_v):
        @pl.when(c + 1 < n_chunks)
        def _():
            gather_start(c + 1, 1 - b)

        pltpu.make_async_copy(
            table_hbm.at[idx_v.at[c]], rows_v, gsems[b]).wait()

        @pl.when(c >= 2)
        def _():
            out_wait(b)

        def group_body(g, gcarry):
            row0 = g * LANES
            rowidx = row0 + iota
            tokv = idx_v[c, pl.ds(row0, LANES)]

            s = jnp.zeros((LANES,), jnp.float32)
            s2 = jnp.zeros((LANES,), jnp.float32)
            for j in range(DIM):
                v = plsc.load_gather(
                    rows_v, [rowidx, jnp.full((LANES,), j, jnp.int32)])
                s = s + v
                s2 = s2 + v * v

            mean = s * (1.0 / DIM)
            var = s2 * (1.0 / DIM) - mean * mean
            inv = _rsqrt(var + EPS)
            m = jnp.where(tokv != 0, 1.0, 0.0).astype(jnp.float32)
            a = inv * m

            scr_v[pl.ds(LANES, LANES)] = mean
            scr_v[pl.ds(2 * LANES, LANES)] = a
            scr_v[pl.ds(3 * LANES, LANES)] = m

            for r in range(LANES):
                mean_b = plsc.load_gather(
                    scr_v, [jnp.full((LANES,), LANES + r, jnp.int32)])
                a_b = plsc.load_gather(
                    scr_v, [jnp.full((LANES,), 2 * LANES + r, jnp.int32)])
                m_b = plsc.load_gather(
                    scr_v, [jnp.full((LANES,), 3 * LANES + r, jnp.int32)])
                row = row0 + r
                xs = [rows_v[row, pl.ds(k * LANES, LANES)] for k in range(K)]
                for k in range(K):
                    scale = a_b * g_regs[k]
                    shift = b_regs[k] * m_b
                    out_v[row, pl.ds(k * LANES, LANES)] = (
                        (xs[k] - mean_b) * scale + shift)
            return gcarry

        lax.fori_loop(0, GROUPS, group_body, 0)
        out_start(c, b)

    def pair_body(cc, carry):
        process(2 * cc, 0, rows0_v, out0_v)
        process(2 * cc + 1, 1, rows1_v, out1_v)
        return carry

    lax.fori_loop(0, n_chunks // 2, pair_body, 0)
    out_wait(0)
    out_wait(1)


def kernel(tokens, table, gamma, beta):
    Bt, Lt = tokens.shape
    N = Bt * Lt
    per_w = N // NW
    n_chunks = per_w // CHUNK
    tok3 = tokens.reshape(NW, n_chunks, CHUNK).astype(jnp.int32)

    mesh = plsc.VectorSubcoreMesh(core_axis_name="c", subcore_axis_name="s")
    sc_call = pl.kernel(
        _body,
        out_type=jax.ShapeDtypeStruct((N, DIM), jnp.float32),
        mesh=mesh,
        compiler_params=pltpu.CompilerParams(
            needs_layout_passes=False, use_tc_tiling_on_sc=False),
        scratch_types=[
            pltpu.VMEM((n_chunks, CHUNK), jnp.int32),
            pltpu.VMEM((CHUNK, DIM), jnp.float32),
            pltpu.VMEM((CHUNK, DIM), jnp.float32),
            pltpu.VMEM((CHUNK, DIM), jnp.float32),
            pltpu.VMEM((CHUNK, DIM), jnp.float32),
            pltpu.VMEM((2 * DIM,), jnp.float32),
            pltpu.VMEM((4 * LANES,), jnp.float32),
            pltpu.SemaphoreType.DMA,
            pltpu.SemaphoreType.DMA,
            pltpu.SemaphoreType.DMA,
            pltpu.SemaphoreType.DMA,
        ],
    )
    out = sc_call(tok3, table, gamma, beta)
    return out.reshape(Bt, Lt, DIM)
